# baseline (speedup 1.0000x reference)
import math

import jax
import jax.numpy as jnp
from jax.experimental import pallas as pl
from jax.experimental.pallas import tpu as pltpu

_B, _UL, _SL, _H = 128, 32, 64, 128
_BC = 32
_CS = _BC * _SL
_NORM = 1.0 / math.sqrt(_H)


def _decode_body(sseq_ref, uin_ref, maskT_ref, wl_ref, s3T_ref, cap0T_ref,
                 Wu_ref, bu_ref, Ws_ref, bs_ref, Wq_ref, bq_ref, Wk_ref, bk_ref,
                 Wv_ref, bv_ref, W1_ref, W2_ref, vt_ref,
                 probs_ref, idx_ref, ua_ref, props_ref,
                 K_ref, V_ref, S1_ref, Q_ref, cap_ref, flag_ref,
                 pacc_ref, iacc_ref, uacc_ref):
    f32 = jnp.float32
    neg = jnp.log(jnp.asarray(1e-45, f32))

    enc0 = jnp.dot(sseq_ref[...].reshape(_CS, 8), Ws_ref[...],
                   preferred_element_type=f32) + bs_ref[...]
    K_ref[...] = (jnp.dot(enc0, Wk_ref[...], preferred_element_type=f32) + bk_ref[...]).reshape(_BC, _SL, _H)
    V_ref[...] = (jnp.dot(enc0, Wv_ref[...], preferred_element_type=f32) + bv_ref[...]).reshape(_BC, _SL, _H)
    S1_ref[...] = jnp.dot(enc0, W1_ref[...], preferred_element_type=f32).reshape(_BC, _SL, _H)
    uenc = jnp.dot(uin_ref[...].reshape(_UL * _BC, 8), Wu_ref[...],
                   preferred_element_type=f32) + bu_ref[...]
    Q_ref[...] = (jnp.dot(uenc, Wq_ref[...], preferred_element_type=f32) + bq_ref[...]).reshape(_UL, _BC, _H)
    cap_ref[...] = cap0T_ref[...]
    flag_ref[...] = jnp.zeros((_BC, _SL), f32)

    vt_col = vt_ref[...]
    iota_s = jax.lax.broadcasted_iota(jnp.int32, (_BC, _SL), 1)
    eye = (jax.lax.broadcasted_iota(jnp.int32, (_BC, _BC), 0)
           == jax.lax.broadcasted_iota(jnp.int32, (_BC, _BC), 1)).astype(f32)

    def body(i, alloc_cnt):
        qT = Q_ref[i].T
        K = K_ref[...]
        E = jnp.dot(K.reshape(_CS, _H), qT, preferred_element_type=f32)
        logits = jnp.sum(E.reshape(_BC, _SL, _BC) * eye[:, None, :], axis=2)
        m = jnp.max(logits, axis=-1, keepdims=True)
        e = jnp.exp(logits - m)
        attn = e / jnp.sum(e, axis=-1, keepdims=True) * _NORM
        Abig = (attn[:, None, :] * eye[:, :, None]).reshape(_BC, _CS)
        g = jnp.dot(Abig, V_ref[...].reshape(_CS, _H), preferred_element_type=f32)
        gW2 = jnp.dot(g, W2_ref[...], preferred_element_type=f32)
        t3 = jnp.tanh(S1_ref[...] + gW2[:, None, :])
        u = jnp.dot(t3.reshape(_CS, _H), vt_col, preferred_element_type=f32).reshape(_BC, _SL)
        mask = maskT_ref[i] > 0.0
        score = jnp.where(mask, u, neg)
        sm = jnp.max(score, axis=-1, keepdims=True)
        ssum = jnp.sum(jnp.exp(score - sm), axis=-1)
        prob = 1.0 / ssum
        idx = jnp.min(jnp.where(score == sm, iota_s, _SL), axis=-1)
        ohf = (iota_s == idx[:, None]).astype(f32)
        w = wl_ref[i].T
        cap = cap_ref[...]
        j_cap = jnp.sum(cap * ohf[None], axis=2)
        caf = jnp.min((j_cap >= w).astype(f32), axis=0)
        upd = w * caf[None]
        cap_ref[...] = cap - ohf[None] * upd[:, :, None]
        oh_eff = ohf * caf[:, None]
        flag_ref[...] = jnp.maximum(flag_ref[...], oh_eff)
        cai = caf.astype(jnp.int32)
        sid = idx * cai + (cai - 1)
        s3 = jnp.sum(s3T_ref[...] * ohf[None], axis=2)
        new_cap = j_cap - upd
        feat = jnp.concatenate([s3, new_cap, jnp.ones((1, _BC), f32)], axis=0)
        row_enc = jax.lax.dot_general(
            feat, Ws_ref[...], (((0,), (0,)), ((), ())),
            preferred_element_type=f32) + bs_ref[...]
        row_K = jnp.dot(row_enc, Wk_ref[...], preferred_element_type=f32) + bk_ref[...]
        row_V = jnp.dot(row_enc, Wv_ref[...], preferred_element_type=f32) + bv_ref[...]
        row_S1 = jnp.dot(row_enc, W1_ref[...], preferred_element_type=f32)
        m3 = oh_eff[:, :, None]
        im3 = 1.0 - m3
        K_ref[...] = K * im3 + row_K[:, None, :] * m3
        V_ref[...] = V_ref[...] * im3 + row_V[:, None, :] * m3
        S1_ref[...] = S1_ref[...] * im3 + row_S1[:, None, :] * m3
        pacc_ref[pl.ds(i, 1), :] = prob[None, :]
        iacc_ref[pl.ds(i, 1), :] = idx[None, :]
        uacc_ref[pl.ds(i, 1), :] = sid[None, :]
        return alloc_cnt + caf

    alloc_cnt = jax.lax.fori_loop(0, _UL, body, jnp.zeros((_BC,), f32))

    uap = alloc_cnt * (1.0 / _UL)
    flag = flag_ref[...]
    sup = jnp.sum(flag, axis=1) * (1.0 / _SL)
    remain = jnp.sum(cap_ref[...] * flag[None], axis=(0, 2))
    used0 = jnp.sum(cap0T_ref[...] * flag[None], axis=(0, 2))
    cup = 1.0 - remain / used0
    obj = -(uap + cup)
    io4 = jax.lax.broadcasted_iota(jnp.int32, (_BC, 4), 1)
    props_ref[...] = jnp.where(
        io4 == 0, obj[:, None],
        jnp.where(io4 == 1, uap[:, None],
                  jnp.where(io4 == 2, sup[:, None], cup[:, None])))
    probs_ref[...] = pacc_ref[...].T
    idx_ref[...] = iacc_ref[...].T
    ua_ref[...] = uacc_ref[...].T


def kernel(user_input_seq, server_input_seq, masks, W_user, b_user, W_server,
           b_server, Wq, bq, Wk, bk, Wv, bv, W1, W2, vt):
    f32 = jnp.float32
    sseq = jnp.concatenate(
        [server_input_seq, jnp.zeros((_B, _SL, 1), f32)], axis=-1)
    user_T = user_input_seq.transpose(1, 0, 2)
    uin = jnp.concatenate(
        [user_T, jnp.zeros((_UL, _B, 2), f32)], axis=-1)
    Wu8 = jnp.concatenate([W_user, jnp.zeros((2, _H), f32)], axis=0)
    maskT = masks.transpose(1, 0, 2).astype(f32)
    wl = user_T[:, :, 2:]
    s3T = server_input_seq[:, :, :3].transpose(2, 0, 1)
    cap0T = server_input_seq[:, :, 3:7].transpose(2, 0, 1)

    f = jax.ShapeDtypeStruct
    out_shape = [
        f((_B, _UL), f32),
        f((_B, _UL), jnp.int32),
        f((_B, _UL), jnp.int32),
        f((_B, 4), f32),
    ]
    grid = (_B // _BC,)
    in_specs = [
        pl.BlockSpec((_BC, _SL, 8), lambda g: (g, 0, 0)),
        pl.BlockSpec((_UL, _BC, 8), lambda g: (0, g, 0)),
        pl.BlockSpec((_UL, _BC, _SL), lambda g: (0, g, 0)),
        pl.BlockSpec((_UL, _BC, 4), lambda g: (0, g, 0)),
        pl.BlockSpec((3, _BC, _SL), lambda g: (0, g, 0)),
        pl.BlockSpec((4, _BC, _SL), lambda g: (0, g, 0)),
        pl.BlockSpec((8, _H), lambda g: (0, 0)),
        pl.BlockSpec((_H,), lambda g: (0,)),
        pl.BlockSpec((8, _H), lambda g: (0, 0)),
        pl.BlockSpec((_H,), lambda g: (0,)),
        pl.BlockSpec((_H, _H), lambda g: (0, 0)),
        pl.BlockSpec((_H,), lambda g: (0,)),
        pl.BlockSpec((_H, _H), lambda g: (0, 0)),
        pl.BlockSpec((_H,), lambda g: (0,)),
        pl.BlockSpec((_H, _H), lambda g: (0, 0)),
        pl.BlockSpec((_H,), lambda g: (0,)),
        pl.BlockSpec((_H, _H), lambda g: (0, 0)),
        pl.BlockSpec((_H, _H), lambda g: (0, 0)),
        pl.BlockSpec((_H, 1), lambda g: (0, 0)),
    ]
    out_specs = [
        pl.BlockSpec((_BC, _UL), lambda g: (g, 0)),
        pl.BlockSpec((_BC, _UL), lambda g: (g, 0)),
        pl.BlockSpec((_BC, _UL), lambda g: (g, 0)),
        pl.BlockSpec((_BC, 4), lambda g: (g, 0)),
    ]
    scratch = [
        pltpu.VMEM((_BC, _SL, _H), f32),
        pltpu.VMEM((_BC, _SL, _H), f32),
        pltpu.VMEM((_BC, _SL, _H), f32),
        pltpu.VMEM((_UL, _BC, _H), f32),
        pltpu.VMEM((4, _BC, _SL), f32),
        pltpu.VMEM((_BC, _SL), f32),
        pltpu.VMEM((_UL, _BC), f32),
        pltpu.VMEM((_UL, _BC), jnp.int32),
        pltpu.VMEM((_UL, _BC), jnp.int32),
    ]
    probsBU, idxBU, uaBU, props = pl.pallas_call(
        _decode_body,
        grid=grid,
        in_specs=in_specs,
        out_specs=out_specs,
        out_shape=out_shape,
        scratch_shapes=scratch,
    )(sseq, uin, maskT, wl, s3T, cap0T,
      Wu8, b_user, W_server, b_server, Wq, bq, Wk, bk, Wv, bv, W1, W2, vt)
    return (props[:, 0], probsBU.T, idxBU, props[:, 1], props[:, 2],
            props[:, 3], uaBU)


# speedup vs baseline: 3.1566x; 3.1566x over previous
import math

import jax
import jax.numpy as jnp
from jax.experimental import pallas as pl
from jax.experimental.pallas import tpu as pltpu

_B, _UL, _SL, _H = 128, 32, 64, 128
_BC = 32
_CS = _BC * _SL
_NORM = 1.0 / math.sqrt(_H)


def _decode_body(sseq_ref, uin_ref, maskT_ref, wl_ref, s3T_ref, cap0T_ref,
                 Wu_ref, bu_ref, Ws_ref, bs_ref, Wq_ref, bq_ref, Wk_ref, bk_ref,
                 Wv_ref, bv_ref, W1_ref, W2_ref, vt_ref,
                 probs_ref, idx_ref, ua_ref, props_ref,
                 K_ref, V_ref, S1_ref, Q_ref, cap_ref, flag_ref,
                 pacc_ref, iacc_ref, uacc_ref):
    f32 = jnp.float32
    neg = jnp.log(jnp.asarray(1e-45, f32))

    enc0 = jnp.dot(sseq_ref[...].reshape(_CS, 8), Ws_ref[...],
                   preferred_element_type=f32) + bs_ref[...]
    K_ref[...] = (jnp.dot(enc0, Wk_ref[...], preferred_element_type=f32) + bk_ref[...]).reshape(_BC, _SL, _H)
    V_ref[...] = (jnp.dot(enc0, Wv_ref[...], preferred_element_type=f32) + bv_ref[...]).reshape(_BC, _SL, _H)
    S1_ref[...] = jnp.dot(enc0, W1_ref[...], preferred_element_type=f32).reshape(_BC, _SL, _H)
    uenc = jnp.dot(uin_ref[...].reshape(_UL * _BC, 8), Wu_ref[...],
                   preferred_element_type=f32) + bu_ref[...]
    Q_ref[...] = (jnp.dot(uenc, Wq_ref[...], preferred_element_type=f32) + bq_ref[...]).reshape(_UL, _BC, _H)
    cap_ref[...] = cap0T_ref[...]
    flag_ref[...] = jnp.zeros((_BC, _SL), f32)

    vt_col = vt_ref[...]
    iota_s = jax.lax.broadcasted_iota(jnp.int32, (_BC, _SL), 1)
    eye = (jax.lax.broadcasted_iota(jnp.int32, (_BC, _BC), 0)
           == jax.lax.broadcasted_iota(jnp.int32, (_BC, _BC), 1)).astype(f32)

    def body(i, alloc_cnt):
        qT = Q_ref[i].T
        K = K_ref[...]
        E = jnp.dot(K.reshape(_CS, _H), qT, preferred_element_type=f32)
        logits = jnp.sum(E.reshape(_BC, _SL, _BC) * eye[:, None, :], axis=2)
        m = jnp.max(logits, axis=-1, keepdims=True)
        e = jnp.exp(logits - m)
        attn = e / jnp.sum(e, axis=-1, keepdims=True) * _NORM
        C = (attn.T[None, :, :] * eye[:, None, :]).reshape(_CS, _BC)
        g = jax.lax.dot_general(
            C, V_ref[...].reshape(_CS, _H), (((0,), (0,)), ((), ())),
            preferred_element_type=f32)
        gW2 = jnp.dot(g, W2_ref[...], preferred_element_type=f32)
        t3 = jnp.tanh(S1_ref[...] + gW2[:, None, :])
        u = jnp.dot(t3.reshape(_CS, _H), vt_col, preferred_element_type=f32).reshape(_BC, _SL)
        mask = maskT_ref[i] > 0.0
        score = jnp.where(mask, u, neg)
        sm = jnp.max(score, axis=-1, keepdims=True)
        ssum = jnp.sum(jnp.exp(score - sm), axis=-1)
        prob = 1.0 / ssum
        idx = jnp.min(jnp.where(score == sm, iota_s, _SL), axis=-1)
        ohf = (iota_s == idx[:, None]).astype(f32)
        w = wl_ref[i].T
        cap = cap_ref[...]
        j_cap = jnp.sum(cap * ohf[None], axis=2)
        caf = jnp.min((j_cap >= w).astype(f32), axis=0)
        upd = w * caf[None]
        cap_ref[...] = cap - ohf[None] * upd[:, :, None]
        oh_eff = ohf * caf[:, None]
        flag_ref[...] = jnp.maximum(flag_ref[...], oh_eff)
        cai = caf.astype(jnp.int32)
        sid = idx * cai + (cai - 1)
        s3 = jnp.sum(s3T_ref[...] * ohf[None], axis=2)
        new_cap = j_cap - upd
        feat = jnp.concatenate([s3, new_cap, jnp.ones((1, _BC), f32)], axis=0)
        row_enc = jax.lax.dot_general(
            feat, Ws_ref[...], (((0,), (0,)), ((), ())),
            preferred_element_type=f32) + bs_ref[...]
        row_K = jnp.dot(row_enc, Wk_ref[...], preferred_element_type=f32) + bk_ref[...]
        row_V = jnp.dot(row_enc, Wv_ref[...], preferred_element_type=f32) + bv_ref[...]
        row_S1 = jnp.dot(row_enc, W1_ref[...], preferred_element_type=f32)
        m3 = oh_eff[:, :, None]
        im3 = 1.0 - m3
        K_ref[...] = K * im3 + row_K[:, None, :] * m3
        V_ref[...] = V_ref[...] * im3 + row_V[:, None, :] * m3
        S1_ref[...] = S1_ref[...] * im3 + row_S1[:, None, :] * m3
        pacc_ref[pl.ds(i, 1), :] = prob[None, :]
        iacc_ref[pl.ds(i, 1), :] = idx[None, :]
        uacc_ref[pl.ds(i, 1), :] = sid[None, :]
        return alloc_cnt + caf

    alloc_cnt = jax.lax.fori_loop(0, _UL, body, jnp.zeros((_BC,), f32))

    uap = alloc_cnt * (1.0 / _UL)
    flag = flag_ref[...]
    sup = jnp.sum(flag, axis=1) * (1.0 / _SL)
    remain = jnp.sum(cap_ref[...] * flag[None], axis=(0, 2))
    used0 = jnp.sum(cap0T_ref[...] * flag[None], axis=(0, 2))
    cup = 1.0 - remain / used0
    obj = -(uap + cup)
    io4 = jax.lax.broadcasted_iota(jnp.int32, (_BC, 4), 1)
    props_ref[...] = jnp.where(
        io4 == 0, obj[:, None],
        jnp.where(io4 == 1, uap[:, None],
                  jnp.where(io4 == 2, sup[:, None], cup[:, None])))
    probs_ref[...] = pacc_ref[...].T
    idx_ref[...] = iacc_ref[...].T
    ua_ref[...] = uacc_ref[...].T


def kernel(user_input_seq, server_input_seq, masks, W_user, b_user, W_server,
           b_server, Wq, bq, Wk, bk, Wv, bv, W1, W2, vt):
    f32 = jnp.float32
    sseq = jnp.concatenate(
        [server_input_seq, jnp.zeros((_B, _SL, 1), f32)], axis=-1)
    user_T = user_input_seq.transpose(1, 0, 2)
    uin = jnp.concatenate(
        [user_T, jnp.zeros((_UL, _B, 2), f32)], axis=-1)
    Wu8 = jnp.concatenate([W_user, jnp.zeros((2, _H), f32)], axis=0)
    maskT = masks.transpose(1, 0, 2).astype(f32)
    wl = user_T[:, :, 2:]
    s3T = server_input_seq[:, :, :3].transpose(2, 0, 1)
    cap0T = server_input_seq[:, :, 3:7].transpose(2, 0, 1)

    f = jax.ShapeDtypeStruct
    out_shape = [
        f((_B, _UL), f32),
        f((_B, _UL), jnp.int32),
        f((_B, _UL), jnp.int32),
        f((_B, 4), f32),
    ]
    grid = (_B // _BC,)
    in_specs = [
        pl.BlockSpec((_BC, _SL, 8), lambda g: (g, 0, 0)),
        pl.BlockSpec((_UL, _BC, 8), lambda g: (0, g, 0)),
        pl.BlockSpec((_UL, _BC, _SL), lambda g: (0, g, 0)),
        pl.BlockSpec((_UL, _BC, 4), lambda g: (0, g, 0)),
        pl.BlockSpec((3, _BC, _SL), lambda g: (0, g, 0)),
        pl.BlockSpec((4, _BC, _SL), lambda g: (0, g, 0)),
        pl.BlockSpec((8, _H), lambda g: (0, 0)),
        pl.BlockSpec((_H,), lambda g: (0,)),
        pl.BlockSpec((8, _H), lambda g: (0, 0)),
        pl.BlockSpec((_H,), lambda g: (0,)),
        pl.BlockSpec((_H, _H), lambda g: (0, 0)),
        pl.BlockSpec((_H,), lambda g: (0,)),
        pl.BlockSpec((_H, _H), lambda g: (0, 0)),
        pl.BlockSpec((_H,), lambda g: (0,)),
        pl.BlockSpec((_H, _H), lambda g: (0, 0)),
        pl.BlockSpec((_H,), lambda g: (0,)),
        pl.BlockSpec((_H, _H), lambda g: (0, 0)),
        pl.BlockSpec((_H, _H), lambda g: (0, 0)),
        pl.BlockSpec((_H, 1), lambda g: (0, 0)),
    ]
    out_specs = [
        pl.BlockSpec((_BC, _UL), lambda g: (g, 0)),
        pl.BlockSpec((_BC, _UL), lambda g: (g, 0)),
        pl.BlockSpec((_BC, _UL), lambda g: (g, 0)),
        pl.BlockSpec((_BC, 4), lambda g: (g, 0)),
    ]
    scratch = [
        pltpu.VMEM((_BC, _SL, _H), f32),
        pltpu.VMEM((_BC, _SL, _H), f32),
        pltpu.VMEM((_BC, _SL, _H), f32),
        pltpu.VMEM((_UL, _BC, _H), f32),
        pltpu.VMEM((4, _BC, _SL), f32),
        pltpu.VMEM((_BC, _SL), f32),
        pltpu.VMEM((_UL, _BC), f32),
        pltpu.VMEM((_UL, _BC), jnp.int32),
        pltpu.VMEM((_UL, _BC), jnp.int32),
    ]
    probsBU, idxBU, uaBU, props = pl.pallas_call(
        _decode_body,
        grid=grid,
        in_specs=in_specs,
        out_specs=out_specs,
        out_shape=out_shape,
        scratch_shapes=scratch,
    )(sseq, uin, maskT, wl, s3T, cap0T,
      Wu8, b_user, W_server, b_server, Wq, bq, Wk, bk, Wv, bv, W1, W2, vt)
    return (props[:, 0], probsBU.T, idxBU, props[:, 1], props[:, 2],
            props[:, 3], uaBU)


# speedup vs baseline: 4.2192x; 1.3366x over previous
import math

import jax
import jax.numpy as jnp
from jax.experimental import pallas as pl
from jax.experimental.pallas import tpu as pltpu

_B, _UL, _SL, _H = 128, 32, 64, 128
_BC = 64
_CS = _BC * _SL
_NORM = 1.0 / math.sqrt(_H)


def _decode_body(sseq_ref, uin_ref, maskT_ref, wl_ref, s3T_ref, cap0T_ref,
                 Wu_ref, bu_ref, Ws_ref, bs_ref, Wq_ref, bq_ref, Wk_ref, bk_ref,
                 Wv_ref, bv_ref, W1_ref, W2_ref, vt_ref,
                 probs_ref, idx_ref, ua_ref, props_ref,
                 K_ref, V_ref, S1_ref, Q_ref, cap_ref, flag_ref,
                 pacc_ref, iacc_ref, uacc_ref):
    f32 = jnp.float32
    neg = jnp.log(jnp.asarray(1e-45, f32))

    enc0 = jnp.dot(sseq_ref[...].reshape(_CS, 8), Ws_ref[...],
                   preferred_element_type=f32) + bs_ref[...]
    K_ref[...] = (jnp.dot(enc0, Wk_ref[...], preferred_element_type=f32) + bk_ref[...]).reshape(_BC, _SL, _H)
    V_ref[...] = (jnp.dot(enc0, Wv_ref[...], preferred_element_type=f32) + bv_ref[...]).reshape(_BC, _SL, _H)
    S1_ref[...] = jnp.dot(enc0, W1_ref[...], preferred_element_type=f32).reshape(_BC, _SL, _H)
    uenc = jnp.dot(uin_ref[...].reshape(_UL * _BC, 8), Wu_ref[...],
                   preferred_element_type=f32) + bu_ref[...]
    Q_ref[...] = (jnp.dot(uenc, Wq_ref[...], preferred_element_type=f32) + bq_ref[...]).reshape(_UL, _BC, _H)
    cap_ref[...] = cap0T_ref[...]
    flag_ref[...] = jnp.zeros((_BC, _SL), f32)

    vt_col = vt_ref[...]
    iota_s = jax.lax.broadcasted_iota(jnp.int32, (_BC, _SL), 1)
    eye = (jax.lax.broadcasted_iota(jnp.int32, (_BC, _BC), 0)
           == jax.lax.broadcasted_iota(jnp.int32, (_BC, _BC), 1)).astype(f32)

    def body(i, alloc_cnt):
        qT = Q_ref[i].T
        K = K_ref[...]
        E = jnp.dot(K.reshape(_CS, _H), qT, preferred_element_type=f32)
        logits = jnp.sum(E.reshape(_BC, _SL, _BC) * eye[:, None, :], axis=2)
        m = jnp.max(logits, axis=-1, keepdims=True)
        e = jnp.exp(logits - m)
        attn = e / jnp.sum(e, axis=-1, keepdims=True) * _NORM
        C = (attn.T[None, :, :] * eye[:, None, :]).reshape(_CS, _BC)
        g = jax.lax.dot_general(
            C, V_ref[...].reshape(_CS, _H), (((0,), (0,)), ((), ())),
            preferred_element_type=f32)
        gW2 = jnp.dot(g, W2_ref[...], preferred_element_type=f32)
        t3 = jnp.tanh(S1_ref[...] + gW2[:, None, :])
        u = jnp.dot(t3.reshape(_CS, _H), vt_col, preferred_element_type=f32).reshape(_BC, _SL)
        mask = maskT_ref[i] > 0.0
        score = jnp.where(mask, u, neg)
        sm = jnp.max(score, axis=-1, keepdims=True)
        ssum = jnp.sum(jnp.exp(score - sm), axis=-1)
        prob = 1.0 / ssum
        idx = jnp.min(jnp.where(score == sm, iota_s, _SL), axis=-1)
        ohf = (iota_s == idx[:, None]).astype(f32)
        w = wl_ref[i].T
        cap = cap_ref[...]
        j_cap = jnp.sum(cap * ohf[None], axis=2)
        caf = jnp.min((j_cap >= w).astype(f32), axis=0)
        upd = w * caf[None]
        cap_ref[...] = cap - ohf[None] * upd[:, :, None]
        oh_eff = ohf * caf[:, None]
        flag_ref[...] = jnp.maximum(flag_ref[...], oh_eff)
        cai = caf.astype(jnp.int32)
        sid = idx * cai + (cai - 1)
        s3 = jnp.sum(s3T_ref[...] * ohf[None], axis=2)
        new_cap = j_cap - upd
        feat = jnp.concatenate([s3, new_cap, jnp.ones((1, _BC), f32)], axis=0)
        row_enc = jax.lax.dot_general(
            feat, Ws_ref[...], (((0,), (0,)), ((), ())),
            preferred_element_type=f32) + bs_ref[...]
        row_K = jnp.dot(row_enc, Wk_ref[...], preferred_element_type=f32) + bk_ref[...]
        row_V = jnp.dot(row_enc, Wv_ref[...], preferred_element_type=f32) + bv_ref[...]
        row_S1 = jnp.dot(row_enc, W1_ref[...], preferred_element_type=f32)
        m3 = oh_eff[:, :, None]
        im3 = 1.0 - m3
        K_ref[...] = K * im3 + row_K[:, None, :] * m3
        V_ref[...] = V_ref[...] * im3 + row_V[:, None, :] * m3
        S1_ref[...] = S1_ref[...] * im3 + row_S1[:, None, :] * m3
        pacc_ref[pl.ds(i, 1), :] = prob[None, :]
        iacc_ref[pl.ds(i, 1), :] = idx[None, :]
        uacc_ref[pl.ds(i, 1), :] = sid[None, :]
        return alloc_cnt + caf

    alloc_cnt = jax.lax.fori_loop(0, _UL, body, jnp.zeros((_BC,), f32))

    uap = alloc_cnt * (1.0 / _UL)
    flag = flag_ref[...]
    sup = jnp.sum(flag, axis=1) * (1.0 / _SL)
    remain = jnp.sum(cap_ref[...] * flag[None], axis=(0, 2))
    used0 = jnp.sum(cap0T_ref[...] * flag[None], axis=(0, 2))
    cup = 1.0 - remain / used0
    obj = -(uap + cup)
    io4 = jax.lax.broadcasted_iota(jnp.int32, (_BC, 4), 1)
    props_ref[...] = jnp.where(
        io4 == 0, obj[:, None],
        jnp.where(io4 == 1, uap[:, None],
                  jnp.where(io4 == 2, sup[:, None], cup[:, None])))
    probs_ref[...] = pacc_ref[...].T
    idx_ref[...] = iacc_ref[...].T
    ua_ref[...] = uacc_ref[...].T


def kernel(user_input_seq, server_input_seq, masks, W_user, b_user, W_server,
           b_server, Wq, bq, Wk, bk, Wv, bv, W1, W2, vt):
    f32 = jnp.float32
    sseq = jnp.concatenate(
        [server_input_seq, jnp.zeros((_B, _SL, 1), f32)], axis=-1)
    user_T = user_input_seq.transpose(1, 0, 2)
    uin = jnp.concatenate(
        [user_T, jnp.zeros((_UL, _B, 2), f32)], axis=-1)
    Wu8 = jnp.concatenate([W_user, jnp.zeros((2, _H), f32)], axis=0)
    maskT = masks.transpose(1, 0, 2).astype(f32)
    wl = user_T[:, :, 2:]
    s3T = server_input_seq[:, :, :3].transpose(2, 0, 1)
    cap0T = server_input_seq[:, :, 3:7].transpose(2, 0, 1)

    f = jax.ShapeDtypeStruct
    out_shape = [
        f((_B, _UL), f32),
        f((_B, _UL), jnp.int32),
        f((_B, _UL), jnp.int32),
        f((_B, 4), f32),
    ]
    grid = (_B // _BC,)
    in_specs = [
        pl.BlockSpec((_BC, _SL, 8), lambda g: (g, 0, 0)),
        pl.BlockSpec((_UL, _BC, 8), lambda g: (0, g, 0)),
        pl.BlockSpec((_UL, _BC, _SL), lambda g: (0, g, 0)),
        pl.BlockSpec((_UL, _BC, 4), lambda g: (0, g, 0)),
        pl.BlockSpec((3, _BC, _SL), lambda g: (0, g, 0)),
        pl.BlockSpec((4, _BC, _SL), lambda g: (0, g, 0)),
        pl.BlockSpec((8, _H), lambda g: (0, 0)),
        pl.BlockSpec((_H,), lambda g: (0,)),
        pl.BlockSpec((8, _H), lambda g: (0, 0)),
        pl.BlockSpec((_H,), lambda g: (0,)),
        pl.BlockSpec((_H, _H), lambda g: (0, 0)),
        pl.BlockSpec((_H,), lambda g: (0,)),
        pl.BlockSpec((_H, _H), lambda g: (0, 0)),
        pl.BlockSpec((_H,), lambda g: (0,)),
        pl.BlockSpec((_H, _H), lambda g: (0, 0)),
        pl.BlockSpec((_H,), lambda g: (0,)),
        pl.BlockSpec((_H, _H), lambda g: (0, 0)),
        pl.BlockSpec((_H, _H), lambda g: (0, 0)),
        pl.BlockSpec((_H, 1), lambda g: (0, 0)),
    ]
    out_specs = [
        pl.BlockSpec((_BC, _UL), lambda g: (g, 0)),
        pl.BlockSpec((_BC, _UL), lambda g: (g, 0)),
        pl.BlockSpec((_BC, _UL), lambda g: (g, 0)),
        pl.BlockSpec((_BC, 4), lambda g: (g, 0)),
    ]
    scratch = [
        pltpu.VMEM((_BC, _SL, _H), f32),
        pltpu.VMEM((_BC, _SL, _H), f32),
        pltpu.VMEM((_BC, _SL, _H), f32),
        pltpu.VMEM((_UL, _BC, _H), f32),
        pltpu.VMEM((4, _BC, _SL), f32),
        pltpu.VMEM((_BC, _SL), f32),
        pltpu.VMEM((_UL, _BC), f32),
        pltpu.VMEM((_UL, _BC), jnp.int32),
        pltpu.VMEM((_UL, _BC), jnp.int32),
    ]
    probsBU, idxBU, uaBU, props = pl.pallas_call(
        _decode_body,
        grid=grid,
        in_specs=in_specs,
        out_specs=out_specs,
        out_shape=out_shape,
        scratch_shapes=scratch,
    )(sseq, uin, maskT, wl, s3T, cap0T,
      Wu8, b_user, W_server, b_server, Wq, bq, Wk, bk, Wv, bv, W1, W2, vt)
    return (props[:, 0], probsBU.T, idxBU, props[:, 1], props[:, 2],
            props[:, 3], uaBU)


# speedup vs baseline: 4.7713x; 1.1309x over previous
import math

import jax
import jax.numpy as jnp
from jax.experimental import pallas as pl
from jax.experimental.pallas import tpu as pltpu

_B, _UL, _SL, _H = 128, 32, 64, 128
_BC = 128
_CS = _BC * _SL
_NORM = 1.0 / math.sqrt(_H)


def _decode_body(sseq_ref, uin_ref, maskT_ref, wl_ref, s3T_ref, cap0T_ref,
                 Wu_ref, bu_ref, Ws_ref, bs_ref, Wq_ref, bq_ref, Wk_ref, bk_ref,
                 Wv_ref, bv_ref, W1_ref, W2_ref, vt_ref,
                 probs_ref, idx_ref, ua_ref, props_ref,
                 K_ref, V_ref, S1_ref, Q_ref, cap_ref, flag_ref,
                 pacc_ref, iacc_ref, uacc_ref):
    f32 = jnp.float32
    neg = jnp.log(jnp.asarray(1e-45, f32))

    enc0 = jnp.dot(sseq_ref[...].reshape(_CS, 8), Ws_ref[...],
                   preferred_element_type=f32) + bs_ref[...]
    K_ref[...] = (jnp.dot(enc0, Wk_ref[...], preferred_element_type=f32) + bk_ref[...]).reshape(_BC, _SL, _H)
    V_ref[...] = (jnp.dot(enc0, Wv_ref[...], preferred_element_type=f32) + bv_ref[...]).reshape(_BC, _SL, _H)
    S1_ref[...] = jnp.dot(enc0, W1_ref[...], preferred_element_type=f32).reshape(_BC, _SL, _H)
    uenc = jnp.dot(uin_ref[...].reshape(_UL * _BC, 8), Wu_ref[...],
                   preferred_element_type=f32) + bu_ref[...]
    Q_ref[...] = (jnp.dot(uenc, Wq_ref[...], preferred_element_type=f32) + bq_ref[...]).reshape(_UL, _BC, _H)
    cap_ref[...] = cap0T_ref[...]
    flag_ref[...] = jnp.zeros((_BC, _SL), f32)

    vt_col = vt_ref[...]
    iota_s = jax.lax.broadcasted_iota(jnp.int32, (_BC, _SL), 1)
    eye = (jax.lax.broadcasted_iota(jnp.int32, (_BC, _BC), 0)
           == jax.lax.broadcasted_iota(jnp.int32, (_BC, _BC), 1)).astype(f32)

    def body(i, alloc_cnt):
        qT = Q_ref[i].T
        K = K_ref[...]
        E = jnp.dot(K.reshape(_CS, _H), qT, preferred_element_type=f32)
        logits = jnp.sum(E.reshape(_BC, _SL, _BC) * eye[:, None, :], axis=2)
        m = jnp.max(logits, axis=-1, keepdims=True)
        e = jnp.exp(logits - m)
        attn = e / jnp.sum(e, axis=-1, keepdims=True) * _NORM
        C = (attn.T[None, :, :] * eye[:, None, :]).reshape(_CS, _BC)
        g = jax.lax.dot_general(
            C, V_ref[...].reshape(_CS, _H), (((0,), (0,)), ((), ())),
            preferred_element_type=f32)
        gW2 = jnp.dot(g, W2_ref[...], preferred_element_type=f32)
        t3 = jnp.tanh(S1_ref[...] + gW2[:, None, :])
        u = jnp.dot(t3.reshape(_CS, _H), vt_col, preferred_element_type=f32).reshape(_BC, _SL)
        mask = maskT_ref[i] > 0.0
        score = jnp.where(mask, u, neg)
        sm = jnp.max(score, axis=-1, keepdims=True)
        ssum = jnp.sum(jnp.exp(score - sm), axis=-1)
        prob = 1.0 / ssum
        idx = jnp.min(jnp.where(score == sm, iota_s, _SL), axis=-1)
        ohf = (iota_s == idx[:, None]).astype(f32)
        w = wl_ref[i].T
        cap = cap_ref[...]
        j_cap = jnp.sum(cap * ohf[None], axis=2)
        caf = jnp.min((j_cap >= w).astype(f32), axis=0)
        upd = w * caf[None]
        cap_ref[...] = cap - ohf[None] * upd[:, :, None]
        oh_eff = ohf * caf[:, None]
        flag_ref[...] = jnp.maximum(flag_ref[...], oh_eff)
        cai = caf.astype(jnp.int32)
        sid = idx * cai + (cai - 1)
        s3 = jnp.sum(s3T_ref[...] * ohf[None], axis=2)
        new_cap = j_cap - upd
        feat = jnp.concatenate([s3, new_cap, jnp.ones((1, _BC), f32)], axis=0)
        row_enc = jax.lax.dot_general(
            feat, Ws_ref[...], (((0,), (0,)), ((), ())),
            preferred_element_type=f32) + bs_ref[...]
        row_K = jnp.dot(row_enc, Wk_ref[...], preferred_element_type=f32) + bk_ref[...]
        row_V = jnp.dot(row_enc, Wv_ref[...], preferred_element_type=f32) + bv_ref[...]
        row_S1 = jnp.dot(row_enc, W1_ref[...], preferred_element_type=f32)
        m3 = oh_eff[:, :, None]
        im3 = 1.0 - m3
        K_ref[...] = K * im3 + row_K[:, None, :] * m3
        V_ref[...] = V_ref[...] * im3 + row_V[:, None, :] * m3
        S1_ref[...] = S1_ref[...] * im3 + row_S1[:, None, :] * m3
        pacc_ref[pl.ds(i, 1), :] = prob[None, :]
        iacc_ref[pl.ds(i, 1), :] = idx[None, :]
        uacc_ref[pl.ds(i, 1), :] = sid[None, :]
        return alloc_cnt + caf

    alloc_cnt = jax.lax.fori_loop(0, _UL, body, jnp.zeros((_BC,), f32))

    uap = alloc_cnt * (1.0 / _UL)
    flag = flag_ref[...]
    sup = jnp.sum(flag, axis=1) * (1.0 / _SL)
    remain = jnp.sum(cap_ref[...] * flag[None], axis=(0, 2))
    used0 = jnp.sum(cap0T_ref[...] * flag[None], axis=(0, 2))
    cup = 1.0 - remain / used0
    obj = -(uap + cup)
    io4 = jax.lax.broadcasted_iota(jnp.int32, (_BC, 4), 1)
    props_ref[...] = jnp.where(
        io4 == 0, obj[:, None],
        jnp.where(io4 == 1, uap[:, None],
                  jnp.where(io4 == 2, sup[:, None], cup[:, None])))
    probs_ref[...] = pacc_ref[...].T
    idx_ref[...] = iacc_ref[...].T
    ua_ref[...] = uacc_ref[...].T


def kernel(user_input_seq, server_input_seq, masks, W_user, b_user, W_server,
           b_server, Wq, bq, Wk, bk, Wv, bv, W1, W2, vt):
    f32 = jnp.float32
    sseq = jnp.concatenate(
        [server_input_seq, jnp.zeros((_B, _SL, 1), f32)], axis=-1)
    user_T = user_input_seq.transpose(1, 0, 2)
    uin = jnp.concatenate(
        [user_T, jnp.zeros((_UL, _B, 2), f32)], axis=-1)
    Wu8 = jnp.concatenate([W_user, jnp.zeros((2, _H), f32)], axis=0)
    maskT = masks.transpose(1, 0, 2).astype(f32)
    wl = user_T[:, :, 2:]
    s3T = server_input_seq[:, :, :3].transpose(2, 0, 1)
    cap0T = server_input_seq[:, :, 3:7].transpose(2, 0, 1)

    f = jax.ShapeDtypeStruct
    out_shape = [
        f((_B, _UL), f32),
        f((_B, _UL), jnp.int32),
        f((_B, _UL), jnp.int32),
        f((_B, 4), f32),
    ]
    grid = (_B // _BC,)
    in_specs = [
        pl.BlockSpec((_BC, _SL, 8), lambda g: (g, 0, 0)),
        pl.BlockSpec((_UL, _BC, 8), lambda g: (0, g, 0)),
        pl.BlockSpec((_UL, _BC, _SL), lambda g: (0, g, 0)),
        pl.BlockSpec((_UL, _BC, 4), lambda g: (0, g, 0)),
        pl.BlockSpec((3, _BC, _SL), lambda g: (0, g, 0)),
        pl.BlockSpec((4, _BC, _SL), lambda g: (0, g, 0)),
        pl.BlockSpec((8, _H), lambda g: (0, 0)),
        pl.BlockSpec((_H,), lambda g: (0,)),
        pl.BlockSpec((8, _H), lambda g: (0, 0)),
        pl.BlockSpec((_H,), lambda g: (0,)),
        pl.BlockSpec((_H, _H), lambda g: (0, 0)),
        pl.BlockSpec((_H,), lambda g: (0,)),
        pl.BlockSpec((_H, _H), lambda g: (0, 0)),
        pl.BlockSpec((_H,), lambda g: (0,)),
        pl.BlockSpec((_H, _H), lambda g: (0, 0)),
        pl.BlockSpec((_H,), lambda g: (0,)),
        pl.BlockSpec((_H, _H), lambda g: (0, 0)),
        pl.BlockSpec((_H, _H), lambda g: (0, 0)),
        pl.BlockSpec((_H, 1), lambda g: (0, 0)),
    ]
    out_specs = [
        pl.BlockSpec((_BC, _UL), lambda g: (g, 0)),
        pl.BlockSpec((_BC, _UL), lambda g: (g, 0)),
        pl.BlockSpec((_BC, _UL), lambda g: (g, 0)),
        pl.BlockSpec((_BC, 4), lambda g: (g, 0)),
    ]
    scratch = [
        pltpu.VMEM((_BC, _SL, _H), f32),
        pltpu.VMEM((_BC, _SL, _H), f32),
        pltpu.VMEM((_BC, _SL, _H), f32),
        pltpu.VMEM((_UL, _BC, _H), f32),
        pltpu.VMEM((4, _BC, _SL), f32),
        pltpu.VMEM((_BC, _SL), f32),
        pltpu.VMEM((_UL, _BC), f32),
        pltpu.VMEM((_UL, _BC), jnp.int32),
        pltpu.VMEM((_UL, _BC), jnp.int32),
    ]
    probsBU, idxBU, uaBU, props = pl.pallas_call(
        _decode_body,
        grid=grid,
        in_specs=in_specs,
        out_specs=out_specs,
        out_shape=out_shape,
        scratch_shapes=scratch,
    )(sseq, uin, maskT, wl, s3T, cap0T,
      Wu8, b_user, W_server, b_server, Wq, bq, Wk, bk, Wv, bv, W1, W2, vt)
    return (props[:, 0], probsBU.T, idxBU, props[:, 1], props[:, 2],
            props[:, 3], uaBU)
